# baseline (speedup 1.0000x reference)
import dataclasses
import functools

import jax
import jax.numpy as jnp
from jax import lax
from jax.experimental import pallas as pl
from jax.experimental.pallas import tpu as pltpu
from jax.experimental.pallas import tpu_sc as plsc

B, L, F = 1024, 20, 26
NUM_FEAT, VOCAB, E = 128, 100000, 32
N = B * L

NC, NS = 2, 16
NW = NC * NS
N_PER_W = N // NW
LANES = 4 * E

VB = 8192
NB = 13
NQ = 7
QLINES = NB * VB


def _sc_compiler_params():
    cp = pltpu.CompilerParams(use_tc_tiling_on_sc=False)
    if "needs_layout_passes" in pltpu.CompilerParams.__dataclass_fields__:
        cp = dataclasses.replace(cp, needs_layout_passes=False)
    return cp


def _tp_body(x_ref, o_ref):
    x4 = x_ref[...].reshape(4 * E, VB)
    o_ref[...] = x4.T


def _transpose_quad(tt, q):
    return pl.pallas_call(
        _tp_body,
        grid=(NB,),
        in_specs=[pl.BlockSpec((4, E, VB), lambda j, q=q: (q, 0, j))],
        out_specs=pl.BlockSpec((VB, LANES), lambda j: (j, 0)),
        out_shape=jax.ShapeDtypeStruct((QLINES, LANES), jnp.float32),
    )(tt)


def _mm_body(x_ref, w_ref, o_ref):
    o_ref[...] = lax.dot_general(
        x_ref[...], w_ref[...],
        (((1,), (1,)), ((), ())),
        preferred_element_type=jnp.float32,
    )


def _matmul(x, w):
    blk = 2048
    return pl.pallas_call(
        _mm_body,
        grid=(N // blk,),
        in_specs=[
            pl.BlockSpec((blk, NUM_FEAT), lambda i: (i, 0)),
            pl.BlockSpec((E, NUM_FEAT), lambda i: (0, 0)),
        ],
        out_specs=pl.BlockSpec((blk, E), lambda i: (i, 0)),
        out_shape=jax.ShapeDtypeStruct((N, E), jnp.float32),
    )(x, w)


def _make_sc_body(fk, gp, ng):
    def _sc_body(table_hbm, lines_hbm, acc_hbm, out_hbm,
                 lines_v, acc_v, rows0, rows1, sem0, sem1):
        cid = lax.axis_index("core")
        sid = lax.axis_index("subcore")
        wid = sid * NC + cid
        base = wid * (N_PER_W * E)

        pltpu.sync_copy(lines_hbm.at[wid], lines_v)
        pltpu.sync_copy(acc_hbm.at[pl.ds(base, N_PER_W * E)], acc_v)

        iota16 = lax.iota(jnp.int32, 16)

        def fire(g, buf, sem):
            pltpu.async_copy(table_hbm.at[lines_v.at[g]], buf, sem)

        def wait(g, buf, sem):
            pltpu.make_async_copy(table_hbm.at[lines_v.at[g]], buf, sem).wait()

        def accum(g, buf):
            for p in range(gp):
                off = (g * gp + p) * E
                a0 = acc_v[pl.ds(off, 16)]
                a1 = acc_v[pl.ds(off + 16, 16)]
                for f in range(fk):
                    r = p * fk + f
                    rvec = jnp.full((16,), r, jnp.int32)
                    col0 = iota16 + (f * E)
                    a0 = a0 + plsc.load_gather(buf, [rvec, col0])
                    a1 = a1 + plsc.load_gather(buf, [rvec, col0 + 16])
                acc_v[pl.ds(off, 16)] = a0
                acc_v[pl.ds(off + 16, 16)] = a1

        fire(0, rows0, sem0)
        fire(1, rows1, sem1)

        @pl.loop(0, ng, step=2)
        def _(g):
            wait(g, rows0, sem0)
            accum(g, rows0)

            @pl.when(g + 2 < ng)
            def _():
                fire(g + 2, rows0, sem0)

            wait(g + 1, rows1, sem1)
            accum(g + 1, rows1)

            @pl.when(g + 3 < ng)
            def _():
                fire(g + 3, rows1, sem1)

        pltpu.sync_copy(acc_v, out_hbm.at[pl.ds(base, N_PER_W * E)])

    return _sc_body


def _sc_pass(table_q, lines_q, acc, fk):
    gp = 128 // fk
    ng = N_PER_W // gp
    mesh = plsc.VectorSubcoreMesh(core_axis_name="core",
                                  subcore_axis_name="subcore")
    sc = pl.kernel(
        _make_sc_body(fk, gp, ng),
        out_type=jax.ShapeDtypeStruct((N * E,), jnp.float32),
        mesh=mesh,
        scratch_types=[
            pltpu.VMEM((ng, 128), jnp.int32),
            pltpu.VMEM((N_PER_W * E,), jnp.float32),
            pltpu.VMEM((128, LANES), jnp.float32),
            pltpu.VMEM((128, LANES), jnp.float32),
            pltpu.SemaphoreType.DMA,
            pltpu.SemaphoreType.DMA,
        ],
        compiler_params=_sc_compiler_params(),
    )
    return sc(table_q, lines_q, acc)


@jax.jit
def kernel(nodes_numerical, nodes_categorical, W_num, tables):
    x = nodes_numerical.reshape(N, NUM_FEAT)
    acc = _matmul(x, W_num).reshape(N * E)

    tt = tables.transpose(0, 2, 1)
    v = nodes_categorical.reshape(N, F)

    for q in range(NQ):
        fk = min(4, F - 4 * q)
        table_q = _transpose_quad(tt, q)
        gp = 128 // fk
        lines_q = v[:, 4 * q:4 * q + fk].reshape(NW, N_PER_W // gp, 128)
        acc = _sc_pass(table_q, lines_q, acc, fk)

    return acc.reshape(B, L, E)


# speedup vs baseline: 1.1683x; 1.1683x over previous
import dataclasses
import functools

import jax
import jax.numpy as jnp
from jax import lax
from jax.experimental import pallas as pl
from jax.experimental.pallas import tpu as pltpu
from jax.experimental.pallas import tpu_sc as plsc

B, L, F = 1024, 20, 26
NUM_FEAT, VOCAB, E = 128, 100000, 32
N = B * L

NC, NS = 2, 16
NW = NC * NS
N_PER_W = N // NW
LANES = 4 * E

VB = 8192
NB = 13
QLINES = NB * VB

SPANS = ((0, 4, 16, 8), (4, 3, 10, 10))


def _sc_compiler_params():
    cp = pltpu.CompilerParams(use_tc_tiling_on_sc=False)
    if "needs_layout_passes" in pltpu.CompilerParams.__dataclass_fields__:
        cp = dataclasses.replace(cp, needs_layout_passes=False)
    return cp


def _tp_body(x_ref, o_ref):
    x4 = x_ref[...].reshape(4 * E, VB)
    o_ref[...] = x4.T


def _transpose_span(tt, q0, nq):
    return pl.pallas_call(
        _tp_body,
        grid=(nq * NB,),
        in_specs=[pl.BlockSpec((4, E, VB),
                               lambda j, q0=q0: (q0 + j // NB, 0, j % NB))],
        out_specs=pl.BlockSpec((VB, LANES), lambda j: (j, 0)),
        out_shape=jax.ShapeDtypeStruct((nq * QLINES, LANES), jnp.float32),
    )(tt)


def _mm_body(x_ref, w_ref, o_ref):
    o_ref[...] = lax.dot_general(
        x_ref[...], w_ref[...],
        (((1,), (1,)), ((), ())),
        preferred_element_type=jnp.float32,
    )


def _matmul(x, w):
    blk = 2048
    return pl.pallas_call(
        _mm_body,
        grid=(N // blk,),
        in_specs=[
            pl.BlockSpec((blk, NUM_FEAT), lambda i: (i, 0)),
            pl.BlockSpec((E, NUM_FEAT), lambda i: (0, 0)),
        ],
        out_specs=pl.BlockSpec((blk, E), lambda i: (i, 0)),
        out_shape=jax.ShapeDtypeStruct((N, E), jnp.float32),
    )(x, w)


def _make_sc_body(fk, gp, ng, gidx):
    def _sc_body(table_hbm, lines_hbm, out_hbm,
                 lines_v, acc_v, rows0, rows1, sem0, sem1):
        cid = lax.axis_index("core")
        sid = lax.axis_index("subcore")
        wid = sid * NC + cid
        base = wid * (N_PER_W * E)

        pltpu.sync_copy(lines_hbm.at[wid], lines_v)

        iota16 = lax.iota(jnp.int32, 16)

        def fire(g, buf, sem):
            pltpu.async_copy(table_hbm.at[lines_v.at[g]], buf, sem)

        def wait(g, buf, sem):
            pltpu.make_async_copy(table_hbm.at[lines_v.at[g]], buf, sem).wait()

        def accum(g, buf):
            for p in range(gp):
                off = (g * gp + p) * E
                a0 = None
                a1 = None
                for f in range(fk):
                    r = p * fk + f
                    rvec = jnp.full((16,), r, jnp.int32)
                    col0 = iota16 + ((f & 3) * E)
                    g0 = plsc.load_gather(buf, [rvec, col0])
                    g1 = plsc.load_gather(buf, [rvec, col0 + 16])
                    a0 = g0 if a0 is None else a0 + g0
                    a1 = g1 if a1 is None else a1 + g1
                acc_v[pl.ds(off, 16)] = a0
                acc_v[pl.ds(off + 16, 16)] = a1

        fire(0, rows0, sem0)
        fire(1, rows1, sem1)

        @pl.loop(0, ng, step=2)
        def _(g):
            wait(g, rows0, sem0)
            accum(g, rows0)

            @pl.when(g + 2 < ng)
            def _():
                fire(g + 2, rows0, sem0)

            wait(g + 1, rows1, sem1)
            accum(g + 1, rows1)

            @pl.when(g + 3 < ng)
            def _():
                fire(g + 3, rows1, sem1)

        pltpu.sync_copy(acc_v, out_hbm.at[pl.ds(base, N_PER_W * E)])

    return _sc_body


def _sc_pass(table_span, lines_span, fk, gp):
    gidx = gp * fk
    ng = N_PER_W // gp
    mesh = plsc.VectorSubcoreMesh(core_axis_name="core",
                                  subcore_axis_name="subcore")
    sc = pl.kernel(
        _make_sc_body(fk, gp, ng, gidx),
        out_type=jax.ShapeDtypeStruct((N * E,), jnp.float32),
        mesh=mesh,
        scratch_types=[
            pltpu.VMEM((ng, gidx), jnp.int32),
            pltpu.VMEM((N_PER_W * E,), jnp.float32),
            pltpu.VMEM((gidx, LANES), jnp.float32),
            pltpu.VMEM((gidx, LANES), jnp.float32),
            pltpu.SemaphoreType.DMA,
            pltpu.SemaphoreType.DMA,
        ],
        compiler_params=_sc_compiler_params(),
    )
    return sc(table_span, lines_span)


@jax.jit
def kernel(nodes_numerical, nodes_categorical, W_num, tables):
    x = nodes_numerical.reshape(N, NUM_FEAT)
    e2 = _matmul(x, W_num)

    tt = tables.transpose(0, 2, 1)
    v = nodes_categorical.reshape(N, F)

    parts = []
    for q0, nq, fk, gp in SPANS:
        table_span = _transpose_span(tt, q0, nq)
        vs = v[:, 4 * q0:4 * q0 + fk]
        qbase = ((jnp.arange(fk, dtype=jnp.int32) >> 2) * QLINES)[None, :]
        lines_span = (vs + qbase).reshape(NW, N_PER_W // gp, gp * fk)
        parts.append(_sc_pass(table_span, lines_span, fk, gp))

    out = e2 + parts[0].reshape(N, E) + parts[1].reshape(N, E)
    return out.reshape(B, L, E)


# speedup vs baseline: 1.1775x; 1.0079x over previous
import dataclasses
import functools

import jax
import jax.numpy as jnp
from jax import lax
from jax.experimental import pallas as pl
from jax.experimental.pallas import tpu as pltpu
from jax.experimental.pallas import tpu_sc as plsc

B, L, F = 1024, 20, 26
NUM_FEAT, VOCAB, E = 128, 100000, 32
N = B * L

NC, NS = 2, 16
NW = NC * NS
N_PER_W = N // NW
LANES = 4 * E

VB = 8192
NB = 13
QLINES = NB * VB

SPANS = ((0, 4, 16, 8), (4, 3, 10, 10))


def _sc_compiler_params():
    cp = pltpu.CompilerParams(use_tc_tiling_on_sc=False)
    if "needs_layout_passes" in pltpu.CompilerParams.__dataclass_fields__:
        cp = dataclasses.replace(cp, needs_layout_passes=False)
    return cp


def _tp_body(x_ref, o_ref):
    x4 = x_ref[...].reshape(4 * E, VB)
    o_ref[...] = x4.T


def _transpose_span(tt, q0, nq):
    return pl.pallas_call(
        _tp_body,
        grid=(nq * NB,),
        in_specs=[pl.BlockSpec((4, E, VB),
                               lambda j, q0=q0: (q0 + j // NB, 0, j % NB))],
        out_specs=pl.BlockSpec((VB, LANES), lambda j: (j, 0)),
        out_shape=jax.ShapeDtypeStruct((nq * QLINES, LANES), jnp.float32),
    )(tt)


def _mm_body(x_ref, w_ref, o_ref):
    o_ref[...] = lax.dot_general(
        x_ref[...], w_ref[...],
        (((1,), (1,)), ((), ())),
        preferred_element_type=jnp.float32,
    )


def _matmul(x, w):
    blk = 2048
    return pl.pallas_call(
        _mm_body,
        grid=(N // blk,),
        in_specs=[
            pl.BlockSpec((blk, NUM_FEAT), lambda i: (i, 0)),
            pl.BlockSpec((E, NUM_FEAT), lambda i: (0, 0)),
        ],
        out_specs=pl.BlockSpec((blk, E), lambda i: (i, 0)),
        out_shape=jax.ShapeDtypeStruct((N, E), jnp.float32),
    )(x, w)


def _make_sc_body(fk, gp, ng, gidx, n_init):
    def _sc_body(*refs):
        table_hbm, lines_hbm = refs[0], refs[1]
        init_hbm = refs[2:2 + n_init]
        out_hbm = refs[2 + n_init]
        lines_v, acc_v = refs[3 + n_init], refs[4 + n_init]
        init_v = refs[5 + n_init:5 + 2 * n_init]
        rows0, rows1, sem0, sem1 = refs[5 + 2 * n_init:]

        cid = lax.axis_index("core")
        sid = lax.axis_index("subcore")
        wid = sid * NC + cid
        base = wid * (N_PER_W * E)

        pltpu.sync_copy(lines_hbm.at[wid], lines_v)
        for src, dst in zip(init_hbm, init_v):
            pltpu.sync_copy(src.at[pl.ds(base, N_PER_W * E)], dst)

        iota16 = lax.iota(jnp.int32, 16)

        def fire(g, buf, sem):
            pltpu.async_copy(table_hbm.at[lines_v.at[g]], buf, sem)

        def wait(g, buf, sem):
            pltpu.make_async_copy(table_hbm.at[lines_v.at[g]], buf, sem).wait()

        def accum(g, buf):
            for p in range(gp):
                off = (g * gp + p) * E
                a0 = None
                a1 = None
                for iv in init_v:
                    i0 = iv[pl.ds(off, 16)]
                    i1 = iv[pl.ds(off + 16, 16)]
                    a0 = i0 if a0 is None else a0 + i0
                    a1 = i1 if a1 is None else a1 + i1
                for f in range(fk):
                    r = p * fk + f
                    rvec = jnp.full((16,), r, jnp.int32)
                    col0 = iota16 + ((f & 3) * E)
                    g0 = plsc.load_gather(buf, [rvec, col0])
                    g1 = plsc.load_gather(buf, [rvec, col0 + 16])
                    a0 = g0 if a0 is None else a0 + g0
                    a1 = g1 if a1 is None else a1 + g1
                acc_v[pl.ds(off, 16)] = a0
                acc_v[pl.ds(off + 16, 16)] = a1

        fire(0, rows0, sem0)
        fire(1, rows1, sem1)

        @pl.loop(0, ng, step=2)
        def _(g):
            wait(g, rows0, sem0)
            accum(g, rows0)

            @pl.when(g + 2 < ng)
            def _():
                fire(g + 2, rows0, sem0)

            wait(g + 1, rows1, sem1)
            accum(g + 1, rows1)

            @pl.when(g + 3 < ng)
            def _():
                fire(g + 3, rows1, sem1)

        pltpu.sync_copy(acc_v, out_hbm.at[pl.ds(base, N_PER_W * E)])

    return _sc_body


def _sc_pass(table_span, lines_span, fk, gp, inits=()):
    gidx = gp * fk
    ng = N_PER_W // gp
    mesh = plsc.VectorSubcoreMesh(core_axis_name="core",
                                  subcore_axis_name="subcore")
    sc = pl.kernel(
        _make_sc_body(fk, gp, ng, gidx, len(inits)),
        out_type=jax.ShapeDtypeStruct((N * E,), jnp.float32),
        mesh=mesh,
        scratch_types=[
            pltpu.VMEM((ng, gidx), jnp.int32),
            pltpu.VMEM((N_PER_W * E,), jnp.float32),
        ] + [
            pltpu.VMEM((N_PER_W * E,), jnp.float32) for _ in inits
        ] + [
            pltpu.VMEM((gidx, LANES), jnp.float32),
            pltpu.VMEM((gidx, LANES), jnp.float32),
            pltpu.SemaphoreType.DMA,
            pltpu.SemaphoreType.DMA,
        ],
        compiler_params=_sc_compiler_params(),
    )
    return sc(table_span, lines_span, *inits)


@jax.jit
def kernel(nodes_numerical, nodes_categorical, W_num, tables):
    x = nodes_numerical.reshape(N, NUM_FEAT)
    e2 = _matmul(x, W_num)

    tt = tables.transpose(0, 2, 1)
    v = nodes_categorical.reshape(N, F)

    def span_inputs(q0, fk, gp):
        vs = v[:, 4 * q0:4 * q0 + fk]
        qbase = ((jnp.arange(fk, dtype=jnp.int32) >> 2) * QLINES)[None, :]
        return (vs + qbase).reshape(NW, N_PER_W // gp, gp * fk)

    (q0a, nqa, fka, gpa), (q0b, nqb, fkb, gpb) = SPANS
    table_a = _transpose_span(tt, q0a, nqa)
    part_a = _sc_pass(table_a, span_inputs(q0a, fka, gpa), fka, gpa)
    table_b = _transpose_span(tt, q0b, nqb)
    out = _sc_pass(table_b, span_inputs(q0b, fkb, gpb), fkb, gpb,
                   inits=(e2.reshape(N * E), part_a))
    return out.reshape(B, L, E)
